# Initial kernel scaffold; baseline (speedup 1.0000x reference)
import jax
import jax.numpy as jnp
from jax.experimental import pallas as pl


def kernel(x, edge_index, W1, b1, W2, b2):
    raise NotImplementedError("write your pallas kernel here")


# speedup vs baseline: 11.6164x; 11.6164x over previous
import functools

import jax
import jax.numpy as jnp
from jax import lax
from jax.experimental import pallas as pl
from jax.experimental.pallas import tpu as pltpu
from jax.experimental.pallas import tpu_sc as plsc

NC = 2
NS = 16
NW = NC * NS
CHUNK = 128
ROWS = 512

_MESH = plsc.VectorSubcoreMesh(core_axis_name="c", subcore_axis_name="s")


def _deg_body(dst_hbm, zs_hbm, ones_hbm, out_hbm, idx_v, ones_v, acc_sh, n_chunks):
    c = lax.axis_index("c")
    s = lax.axis_index("s")
    wid = c * NS + s

    @pl.when(s == 0)
    def _():
        pltpu.sync_copy(zs_hbm, acc_sh)

    pltpu.sync_copy(dst_hbm.at[wid], idx_v)
    pltpu.sync_copy(ones_hbm, ones_v)
    plsc.subcore_barrier()

    def chunk(i, carry):
        pltpu.sync_copy(ones_v, acc_sh.at[idx_v.at[i]], add=True)
        return carry

    lax.fori_loop(0, n_chunks, chunk, 0, unroll=False)
    plsc.subcore_barrier()

    @pl.when(s == 0)
    def _():
        pltpu.sync_copy(acc_sh, out_hbm.at[c])


def _agg_body(src_hbm, dst_hbm, y_hbm, zs_hbm, out_hbm,
              src_v, dst_v, rows_v, acc_sh, sem, n_chunks):
    c = lax.axis_index("c")
    s = lax.axis_index("s")
    wid = c * NS + s

    @pl.when(s == 0)
    def _():
        pltpu.sync_copy(zs_hbm, acc_sh)

    pltpu.sync_copy(src_hbm.at[wid], src_v)
    pltpu.sync_copy(dst_hbm.at[wid], dst_v)
    plsc.subcore_barrier()

    def chunk(i, carry):
        pltpu.async_copy(y_hbm.at[src_v.at[i]], rows_v, sem).wait()
        pltpu.sync_copy(rows_v, acc_sh.at[dst_v.at[i]], add=True)
        return carry

    lax.fori_loop(0, n_chunks, chunk, 0, unroll=False)
    plsc.subcore_barrier()

    @pl.when(s == 0)
    def _():
        pltpu.sync_copy(acc_sh, out_hbm.at[c])


def _mm1_body(x_ref, w_ref, degp_ref, y_ref, dinv_ref):
    deg = (jnp.sum(degp_ref[0] + degp_ref[1], axis=1, keepdims=True)
           * (1.0 / degp_ref.shape[-1]) + 1.0)
    dinv = lax.rsqrt(deg)
    dinv_ref[...] = dinv
    y_ref[...] = dinv * jnp.dot(x_ref[...], w_ref[...],
                                preferred_element_type=jnp.float32)


def _mm2_body(p_ref, y_ref, dinv_ref, b_ref, w_ref, y2_ref):
    dinv = dinv_ref[...]
    h = dinv * (p_ref[0] + p_ref[1] + y_ref[...]) + b_ref[...]
    h = jnp.maximum(h, 0.0)
    y2_ref[...] = dinv * jnp.dot(h, w_ref[...],
                                 preferred_element_type=jnp.float32)


def _fin_body(p_ref, y_ref, dinv_ref, b_ref, out_ref):
    out_ref[...] = dinv_ref[...] * (p_ref[0] + p_ref[1] + y_ref[...]) + b_ref[...]


def kernel(x, edge_index, W1, b1, W2, b2):
    n, d = x.shape
    e = edge_index.shape[1]
    n_pad = pl.cdiv(n, ROWS) * ROWS
    n_chunks = pl.cdiv(e, NW * CHUNK)
    e_pad = NW * n_chunks * CHUNK

    src = edge_index[0].astype(jnp.int32)
    dst = edge_index[1].astype(jnp.int32)
    pad = e_pad - e
    src3 = jnp.concatenate([src, jnp.zeros((pad,), jnp.int32)]
                           ).reshape(NW, n_chunks, CHUNK)
    dst3 = jnp.concatenate([dst, jnp.full((pad,), n_pad - 1, jnp.int32)]
                           ).reshape(NW, n_chunks, CHUNK)
    x_pad = jnp.pad(x, ((0, n_pad - n), (0, 0)))
    ones2 = jnp.ones((CHUNK, d), jnp.float32)
    zs = jnp.zeros((n_pad, d), jnp.float32)
    b1r = b1.reshape(1, d)
    b2r = b2.reshape(1, d)

    deg_parts = pl.kernel(
        functools.partial(_deg_body, n_chunks=n_chunks),
        out_type=jax.ShapeDtypeStruct((NC, n_pad, d), jnp.float32),
        mesh=_MESH,
        scratch_types=[
            pltpu.VMEM((n_chunks, CHUNK), jnp.int32),
            pltpu.VMEM((CHUNK, d), jnp.float32),
            pltpu.VMEM_SHARED((n_pad, d), jnp.float32),
        ],
    )(dst3, zs, ones2)

    agg = pl.kernel(
        functools.partial(_agg_body, n_chunks=n_chunks),
        out_type=jax.ShapeDtypeStruct((NC, n_pad, d), jnp.float32),
        mesh=_MESH,
        scratch_types=[
            pltpu.VMEM((n_chunks, CHUNK), jnp.int32),
            pltpu.VMEM((n_chunks, CHUNK), jnp.int32),
            pltpu.VMEM((CHUNK, d), jnp.float32),
            pltpu.VMEM_SHARED((n_pad, d), jnp.float32),
            pltpu.SemaphoreType.DMA,
        ],
    )

    grid = n_pad // ROWS
    row_blk = pl.BlockSpec((ROWS, d), lambda i: (i, 0))
    col_blk = pl.BlockSpec((ROWS, 1), lambda i: (i, 0))
    w_blk = pl.BlockSpec((d, d), lambda i: (0, 0))
    b_blk = pl.BlockSpec((1, d), lambda i: (0, 0))
    parts_blk_d = pl.BlockSpec((NC, ROWS, d), lambda i: (0, i, 0))

    y1, dinv = pl.pallas_call(
        _mm1_body,
        grid=(grid,),
        in_specs=[row_blk, w_blk, parts_blk_d],
        out_specs=[row_blk, col_blk],
        out_shape=[jax.ShapeDtypeStruct((n_pad, d), jnp.float32),
                   jax.ShapeDtypeStruct((n_pad, 1), jnp.float32)],
    )(x_pad, W1, deg_parts)

    parts1 = agg(src3, dst3, y1, zs)

    y2 = pl.pallas_call(
        _mm2_body,
        grid=(grid,),
        in_specs=[parts_blk_d, row_blk, col_blk, b_blk, w_blk],
        out_specs=row_blk,
        out_shape=jax.ShapeDtypeStruct((n_pad, d), jnp.float32),
    )(parts1, y1, dinv, b1r, W2)

    parts2 = agg(src3, dst3, y2, zs)

    out = pl.pallas_call(
        _fin_body,
        grid=(grid,),
        in_specs=[parts_blk_d, row_blk, col_blk, b_blk],
        out_specs=row_blk,
        out_shape=jax.ShapeDtypeStruct((n_pad, d), jnp.float32),
    )(parts2, y2, dinv, b2r)

    return out[:n]
